# speedup vs baseline: 1.6211x; 1.6211x over previous
import functools

import jax
import jax.numpy as jnp
from jax import lax
from jax.experimental import pallas as pl
from jax.experimental.pallas import tpu as pltpu
from jax.experimental.pallas import tpu_sc as plsc

BS, MAXLEN, HS = 16, 512, 1024
N_SCHEMA, N_COPY = 128, 384
NSCH = BS * N_SCHEMA
NCP = BS * N_COPY
NW = 32
CH = 64

_SCH_PER_W = NSCH // NW
_CP_PER_W = NCP // NW

_mesh = plsc.VectorSubcoreMesh(core_axis_name="c", subcore_axis_name="s")


@functools.partial(
    pl.kernel,
    mesh=_mesh,
    out_type=[
        jax.ShapeDtypeStruct((NSCH, HS), jnp.float32),
        jax.ShapeDtypeStruct((NCP, HS), jnp.float32),
    ],
    scratch_types=[
        pltpu.VMEM((CH,), jnp.int32),
        pltpu.VMEM((CH, HS), jnp.float32),
        pltpu.SemaphoreType.DMA,
    ],
)
def _compact_rows(flat_hbm, sidx_hbm, cidx_hbm, schema_hbm, copy_hbm,
                  idx_v, rows_v, sem):
    wid = lax.axis_index("s") * 2 + lax.axis_index("c")
    base = wid * _SCH_PER_W
    pltpu.sync_copy(sidx_hbm.at[pl.ds(base, CH)], idx_v)
    pltpu.async_copy(flat_hbm.at[idx_v], rows_v, sem).wait()
    pltpu.sync_copy(rows_v, schema_hbm.at[pl.ds(base, CH)])
    for t in range(_CP_PER_W // CH):
        cbase = wid * _CP_PER_W + t * CH
        pltpu.sync_copy(cidx_hbm.at[pl.ds(cbase, CH)], idx_v)
        pltpu.async_copy(flat_hbm.at[idx_v], rows_v, sem).wait()
        pltpu.sync_copy(rows_v, copy_hbm.at[pl.ds(cbase, CH)])


def kernel(inputs, mask, select_schema_mask, schema_mask, select_copy_mask,
           copy_mask, copy_ids, word_embed):
    flat = inputs.reshape(-1, HS)
    sidx = jnp.nonzero(select_schema_mask.reshape(-1), size=NSCH,
                       fill_value=0)[0].astype(jnp.int32)
    cidx = jnp.nonzero(select_copy_mask.reshape(-1), size=NCP,
                       fill_value=0)[0].astype(jnp.int32)
    schema_flat, copy_flat = _compact_rows(flat, sidx, cidx)
    return (inputs,
            schema_flat.reshape(BS, N_SCHEMA, HS),
            copy_flat.reshape(BS, N_COPY, HS),
            word_embed)


# speedup vs baseline: 1.6281x; 1.0043x over previous
import functools

import jax
import jax.numpy as jnp
from jax import lax
from jax.experimental import pallas as pl
from jax.experimental.pallas import tpu as pltpu
from jax.experimental.pallas import tpu_sc as plsc

BS, MAXLEN, HS = 16, 512, 1024
N_SCHEMA, N_COPY = 128, 384
NSCH = BS * N_SCHEMA
NCP = BS * N_COPY
NW = 32
CH = 32

_SCH_PER_W = NSCH // NW
_CP_PER_W = NCP // NW
_ROWS_PER_W = _SCH_PER_W + _CP_PER_W
_NCH = _ROWS_PER_W // CH
_SCH_CH = _SCH_PER_W // CH

_mesh = plsc.VectorSubcoreMesh(core_axis_name="c", subcore_axis_name="s")


@functools.partial(
    pl.kernel,
    mesh=_mesh,
    out_type=[
        jax.ShapeDtypeStruct((NSCH, HS), jnp.float32),
        jax.ShapeDtypeStruct((NCP, HS), jnp.float32),
    ],
    scratch_types=[
        pltpu.VMEM((_ROWS_PER_W,), jnp.int32),
        pltpu.VMEM((CH, HS), jnp.float32),
        pltpu.VMEM((CH, HS), jnp.float32),
        pltpu.SemaphoreType.DMA,
        pltpu.SemaphoreType.DMA,
        pltpu.SemaphoreType.DMA,
        pltpu.SemaphoreType.DMA,
        pltpu.SemaphoreType.DMA,
    ],
)
def _compact_rows(flat_hbm, sidx_hbm, cidx_hbm, schema_hbm, copy_hbm,
                  idx_v, buf0, buf1, isem, gsem0, gsem1, ssem0, ssem1):
    wid = lax.axis_index("s") * 2 + lax.axis_index("c")
    bufs = (buf0, buf1)
    gsems = (gsem0, gsem1)
    ssems = (ssem0, ssem1)

    ld_s = pltpu.async_copy(sidx_hbm.at[pl.ds(wid * _SCH_PER_W, _SCH_PER_W)],
                            idx_v.at[pl.ds(0, _SCH_PER_W)], isem)
    ld_c = pltpu.async_copy(cidx_hbm.at[pl.ds(wid * _CP_PER_W, _CP_PER_W)],
                            idx_v.at[pl.ds(_SCH_PER_W, _CP_PER_W)], isem)
    ld_s.wait()
    ld_c.wait()

    def _gather(k):
        return pltpu.async_copy(
            flat_hbm.at[idx_v.at[pl.ds(k * CH, CH)]], bufs[k % 2],
            gsems[k % 2])

    def _store(k):
        if k < _SCH_CH:
            dst = schema_hbm.at[pl.ds(wid * _SCH_PER_W + k * CH, CH)]
        else:
            dst = copy_hbm.at[
                pl.ds(wid * _CP_PER_W + (k - _SCH_CH) * CH, CH)]
        return pltpu.async_copy(bufs[k % 2], dst, ssems[k % 2])

    gathers = [None] * _NCH
    stores = [None] * _NCH
    gathers[0] = _gather(0)
    for k in range(_NCH):
        if k + 1 < _NCH:
            if k >= 1:
                stores[k - 1].wait()
            gathers[k + 1] = _gather(k + 1)
        gathers[k].wait()
        stores[k] = _store(k)
    stores[_NCH - 2].wait()
    stores[_NCH - 1].wait()


def kernel(inputs, mask, select_schema_mask, schema_mask, select_copy_mask,
           copy_mask, copy_ids, word_embed):
    flat = inputs.reshape(-1, HS)
    sidx = jnp.nonzero(select_schema_mask.reshape(-1), size=NSCH,
                       fill_value=0)[0].astype(jnp.int32)
    cidx = jnp.nonzero(select_copy_mask.reshape(-1), size=NCP,
                       fill_value=0)[0].astype(jnp.int32)
    schema_flat, copy_flat = _compact_rows(flat, sidx, cidx)
    return (inputs,
            schema_flat.reshape(BS, N_SCHEMA, HS),
            copy_flat.reshape(BS, N_COPY, HS),
            word_embed)
